# baseline (speedup 1.0000x reference)
import functools

import jax
import jax.numpy as jnp
from jax import lax
from jax.experimental import pallas as pl
from jax.experimental.pallas import tpu as pltpu
from jax.experimental.pallas import tpu_sc as plsc

N = 1000000
MD = 64
DECAY = 0.1
B = 16384

NC = 2
NS = 16
NW = NC * NS
BPW = B // NW
CHUNK = 128
CPT = BPW // CHUNK
IDX_ROWS = B // CHUNK

def _sc_gather_body(mem_hbm, lu_hbm, sidx_hbm, didx_hbm,
                    out_ms, out_md, out_ls, out_ld,
                    idx_s, idx_d, rows_s, rows_d, lus, lud, sem):
    wid = lax.axis_index("s") * NC + lax.axis_index("c")
    crow = wid * CPT
    base = wid * BPW
    pltpu.sync_copy(sidx_hbm.at[pl.ds(crow, CPT), :], idx_s)
    pltpu.sync_copy(didx_hbm.at[pl.ds(crow, CPT), :], idx_d)
    copies = []
    for k in range(CPT):
        copies.append(pltpu.make_async_copy(
            mem_hbm.at[idx_s.at[k]], rows_s.at[pl.ds(k * CHUNK, CHUNK), :], sem))
        copies.append(pltpu.make_async_copy(
            mem_hbm.at[idx_d.at[k]], rows_d.at[pl.ds(k * CHUNK, CHUNK), :], sem))
        copies.append(pltpu.make_async_copy(lu_hbm.at[idx_s.at[k]], lus.at[k], sem))
        copies.append(pltpu.make_async_copy(lu_hbm.at[idx_d.at[k]], lud.at[k], sem))
    for c in copies:
        c.start()
    for c in copies:
        c.wait()
    pltpu.sync_copy(rows_s, out_ms.at[pl.ds(base, BPW), :])
    pltpu.sync_copy(rows_d, out_md.at[pl.ds(base, BPW), :])
    pltpu.sync_copy(lus, out_ls.at[pl.ds(crow, CPT), :])
    pltpu.sync_copy(lud, out_ld.at[pl.ds(crow, CPT), :])


@functools.cache
def _make_sc_gather():
    mesh = plsc.VectorSubcoreMesh(
        core_axis_name="c", subcore_axis_name="s",
        num_cores=NC, num_subcores=NS)
    return pl.kernel(
        _sc_gather_body,
        out_type=[
            jax.ShapeDtypeStruct((B, MD), jnp.float32),
            jax.ShapeDtypeStruct((B, MD), jnp.float32),
            jax.ShapeDtypeStruct((IDX_ROWS, CHUNK), jnp.float32),
            jax.ShapeDtypeStruct((IDX_ROWS, CHUNK), jnp.float32),
        ],
        mesh=mesh,
        scratch_types=[
            pltpu.VMEM((CPT, CHUNK), jnp.int32),
            pltpu.VMEM((CPT, CHUNK), jnp.int32),
            pltpu.VMEM((BPW, MD), jnp.float32),
            pltpu.VMEM((BPW, MD), jnp.float32),
            pltpu.VMEM((CPT, CHUNK), jnp.float32),
            pltpu.VMEM((CPT, CHUNK), jnp.float32),
            pltpu.SemaphoreType.DMA,
        ],
        compiler_params=pltpu.CompilerParams(use_tc_tiling_on_sc=False),
    )


def _sc_gather(memory, last_update, sidx, didx):
    return _make_sc_gather()(memory, last_update, sidx, didx)


def _tc_dense_body(ts_ref, lus_ref, lud_ref, ms_ref, md_ref, sf_ref, df_ref,
                   Wn_ref, bn_ref, Wp1_ref, bp1_ref, Wp2_ref, bp2_ref,
                   Wl1_ref, bl1_ref, Wl2_ref, bl2_ref,
                   link_ref, semb_ref, demb_ref):
    ts = ts_ref[...]

    def emb(lu_ref, m_ref, f_ref):
        dt = jnp.maximum(ts - lu_ref[...], 0.0)
        s2 = jnp.exp((-2.0 * DECAY) * dt)
        att = m_ref[...] * s2
        node = jnp.dot(f_ref[...], Wn_ref[...],
                       preferred_element_type=jnp.float32) + bn_ref[...]
        h = jnp.maximum(
            jnp.dot(att, Wp1_ref[0:MD, :], preferred_element_type=jnp.float32)
            + jnp.dot(node, Wp1_ref[MD:2 * MD, :],
                      preferred_element_type=jnp.float32)
            + bp1_ref[...], 0.0)
        return jnp.dot(h, Wp2_ref[...],
                       preferred_element_type=jnp.float32) + bp2_ref[...]

    es = emb(lus_ref, ms_ref, sf_ref)
    ed = emb(lud_ref, md_ref, df_ref)
    semb_ref[...] = es
    demb_ref[...] = ed
    hl = jnp.maximum(
        jnp.dot(es, Wl1_ref[0:MD, :], preferred_element_type=jnp.float32)
        + jnp.dot(ed, Wl1_ref[MD:2 * MD, :], preferred_element_type=jnp.float32)
        + bl1_ref[...], 0.0)
    logit = jnp.dot(hl, Wl2_ref[...],
                    preferred_element_type=jnp.float32) + bl2_ref[...]
    link_ref[...] = 1.0 / (1.0 + jnp.exp(-logit))


_TC_BLK = 2048


def _tc_dense(ts2, lus2, lud2, mem_s, mem_d, sf, df,
              Wn, bn2, Wp1, bp12, Wp2, bp22, Wl1, bl12, Wl2, bl22):
    nb = B // _TC_BLK
    col = pl.BlockSpec((_TC_BLK, 1), lambda i: (i, 0))
    m64 = pl.BlockSpec((_TC_BLK, MD), lambda i: (i, 0))
    f128 = pl.BlockSpec((_TC_BLK, 128), lambda i: (i, 0))

    def w(shape):
        return pl.BlockSpec(shape, lambda i: (0, 0))

    return pl.pallas_call(
        _tc_dense_body,
        grid=(nb,),
        in_specs=[col, col, col, m64, m64, f128, f128,
                  w((128, MD)), w((1, MD)), w((128, MD)), w((1, MD)),
                  w((MD, MD)), w((1, MD)), w((128, MD)), w((1, MD)),
                  w((MD, 1)), w((1, 1))],
        out_specs=[col, m64, m64],
        out_shape=[
            jax.ShapeDtypeStruct((B, 1), jnp.float32),
            jax.ShapeDtypeStruct((B, MD), jnp.float32),
            jax.ShapeDtypeStruct((B, MD), jnp.float32),
        ],
    )(ts2, lus2, lud2, mem_s, mem_d, sf, df,
      Wn, bn2, Wp1, bp12, Wp2, bp22, Wl1, bl12, Wl2, bl22)


def kernel(src_ids, dst_ids, src_features, dst_features, timestamps,
           edge_features, memory, last_update, Wn, bn, We, be, Wt, bt,
           Wp1, bp1, Wp2, bp2, Wl1, bl1, Wl2, bl2):
    sidx = src_ids.astype(jnp.int32).reshape(IDX_ROWS, CHUNK)
    didx = dst_ids.astype(jnp.int32).reshape(IDX_ROWS, CHUNK)
    mem_s, mem_d, lus, lud = _sc_gather(memory, last_update, sidx, didx)
    ts2 = timestamps.reshape(B, 1)
    lus2 = lus.reshape(B, 1)
    lud2 = lud.reshape(B, 1)
    link, semb, demb = _tc_dense(
        ts2, lus2, lud2, mem_s, mem_d, src_features, dst_features,
        Wn, bn.reshape(1, MD), Wp1, bp1.reshape(1, MD), Wp2, bp2.reshape(1, MD),
        Wl1, bl1.reshape(1, MD), Wl2, bl2.reshape(1, 1))
    return link, semb, demb


# speedup vs baseline: 1.0083x; 1.0083x over previous
import functools

import jax
import jax.numpy as jnp
from jax import lax
from jax.experimental import pallas as pl
from jax.experimental.pallas import tpu as pltpu
from jax.experimental.pallas import tpu_sc as plsc

N = 1000000
MD = 64
DECAY = 0.1
B = 16384

NC = 2
NS = 16
NW = NC * NS
BPW = B // NW
CHUNK = 128
CPT = BPW // CHUNK
IDX_ROWS = B // CHUNK
NPAIR = N // 2

_PACK_ROWS = 4000
_PACK_GRID = NPAIR // _PACK_ROWS


def _repack_body(lo_ref, hi_ref, out_ref):
    out_ref[:, 0:MD] = lo_ref[...]
    out_ref[:, MD:2 * MD] = hi_ref[...]


def _repack(memory):
    return pl.pallas_call(
        _repack_body,
        grid=(_PACK_GRID,),
        in_specs=[pl.BlockSpec((_PACK_ROWS, MD), lambda i: (i, 0)),
                  pl.BlockSpec((_PACK_ROWS, MD), lambda i: (i + _PACK_GRID, 0))],
        out_specs=pl.BlockSpec((_PACK_ROWS, 2 * MD), lambda i: (i, 0)),
        out_shape=jax.ShapeDtypeStruct((NPAIR, 2 * MD), jnp.float32),
    )(memory, memory)


def _sc_gather_body(mem_hbm, lu_hbm, spair_hbm, dpair_hbm, sidx_hbm, didx_hbm,
                    out_gs, out_gd, out_ls, out_ld,
                    tid, idx_s, idx_d, rows, lus, lud, sem):
    wid = lax.axis_index("s") * NC + lax.axis_index("c")
    crow = wid * CPT
    base = wid * BPW
    pltpu.sync_copy(sidx_hbm.at[pl.ds(crow, CPT), :], idx_s)
    pltpu.sync_copy(didx_hbm.at[pl.ds(crow, CPT), :], idx_d)
    lu_copies = []
    for k in range(CPT):
        lu_copies.append(pltpu.make_async_copy(
            lu_hbm.at[idx_s.at[k]], lus.at[k], sem))
        lu_copies.append(pltpu.make_async_copy(
            lu_hbm.at[idx_d.at[k]], lud.at[k], sem))
    for c in lu_copies:
        c.start()
    for pair_hbm, out_ref in ((spair_hbm, out_gs), (dpair_hbm, out_gd)):
        pltpu.sync_copy(pair_hbm.at[pl.ds(crow, CPT), :], tid)
        copies = []
        for k in range(CPT):
            copies.append(pltpu.make_async_copy(
                mem_hbm.at[tid.at[k]], rows.at[pl.ds(k * CHUNK, CHUNK), :], sem))
        for c in copies:
            c.start()
        for c in copies:
            c.wait()
        pltpu.sync_copy(rows, out_ref.at[pl.ds(base, BPW), :])
    for c in lu_copies:
        c.wait()
    pltpu.sync_copy(lus, out_ls.at[pl.ds(crow, CPT), :])
    pltpu.sync_copy(lud, out_ld.at[pl.ds(crow, CPT), :])


@functools.cache
def _make_sc_gather():
    mesh = plsc.VectorSubcoreMesh(
        core_axis_name="c", subcore_axis_name="s",
        num_cores=NC, num_subcores=NS)
    return pl.kernel(
        _sc_gather_body,
        out_type=[
            jax.ShapeDtypeStruct((B, 2 * MD), jnp.float32),
            jax.ShapeDtypeStruct((B, 2 * MD), jnp.float32),
            jax.ShapeDtypeStruct((IDX_ROWS, CHUNK), jnp.float32),
            jax.ShapeDtypeStruct((IDX_ROWS, CHUNK), jnp.float32),
        ],
        mesh=mesh,
        scratch_types=[
            pltpu.VMEM((CPT, CHUNK), jnp.int32),
            pltpu.VMEM((CPT, CHUNK), jnp.int32),
            pltpu.VMEM((CPT, CHUNK), jnp.int32),
            pltpu.VMEM((BPW, 2 * MD), jnp.float32),
            pltpu.VMEM((CPT, CHUNK), jnp.float32),
            pltpu.VMEM((CPT, CHUNK), jnp.float32),
            pltpu.SemaphoreType.DMA,
        ],
        compiler_params=pltpu.CompilerParams(use_tc_tiling_on_sc=False),
    )


def _sc_gather(memL, last_update, spair, dpair, sidx, didx):
    return _make_sc_gather()(memL, last_update, spair, dpair, sidx, didx)


def _tc_dense_body(ts_ref, lus_ref, lud_ref, rs_ref, rd_ref,
                   gs_ref, gd_ref, sf_ref, df_ref,
                   Wn_ref, bn_ref, Wp1_ref, bp1_ref, Wp2_ref, bp2_ref,
                   Wl1_ref, bl1_ref, Wl2_ref, bl2_ref,
                   link_ref, semb_ref, demb_ref):
    ts = ts_ref[...]

    def emb(lu_ref, r_ref, g_ref, f_ref):
        r = r_ref[...]
        g = g_ref[...]
        lo = (r == 0).astype(jnp.float32)
        hi = (r == 1).astype(jnp.float32)
        mem = g[:, 0:MD] * lo + g[:, MD:2 * MD] * hi
        dt = jnp.maximum(ts - lu_ref[...], 0.0)
        s2 = jnp.exp((-2.0 * DECAY) * dt)
        att = mem * s2
        node = jnp.dot(f_ref[...], Wn_ref[...],
                       preferred_element_type=jnp.float32) + bn_ref[...]
        h = jnp.maximum(
            jnp.dot(att, Wp1_ref[0:MD, :], preferred_element_type=jnp.float32)
            + jnp.dot(node, Wp1_ref[MD:2 * MD, :],
                      preferred_element_type=jnp.float32)
            + bp1_ref[...], 0.0)
        return jnp.dot(h, Wp2_ref[...],
                       preferred_element_type=jnp.float32) + bp2_ref[...]

    es = emb(lus_ref, rs_ref, gs_ref, sf_ref)
    ed = emb(lud_ref, rd_ref, gd_ref, df_ref)
    semb_ref[...] = es
    demb_ref[...] = ed
    hl = jnp.maximum(
        jnp.dot(es, Wl1_ref[0:MD, :], preferred_element_type=jnp.float32)
        + jnp.dot(ed, Wl1_ref[MD:2 * MD, :], preferred_element_type=jnp.float32)
        + bl1_ref[...], 0.0)
    logit = jnp.dot(hl, Wl2_ref[...],
                    preferred_element_type=jnp.float32) + bl2_ref[...]
    link_ref[...] = 1.0 / (1.0 + jnp.exp(-logit))


_TC_BLK = 2048


def _tc_dense(ts2, lus2, lud2, rs2, rd2, gs, gd, sf, df,
              Wn, bn2, Wp1, bp12, Wp2, bp22, Wl1, bl12, Wl2, bl22):
    nb = B // _TC_BLK
    col = pl.BlockSpec((_TC_BLK, 1), lambda i: (i, 0))
    m64 = pl.BlockSpec((_TC_BLK, MD), lambda i: (i, 0))
    f128 = pl.BlockSpec((_TC_BLK, 128), lambda i: (i, 0))

    def w(shape):
        return pl.BlockSpec(shape, lambda i: (0, 0))

    return pl.pallas_call(
        _tc_dense_body,
        grid=(nb,),
        in_specs=[col, col, col, col, col, f128, f128, f128, f128,
                  w((128, MD)), w((1, MD)), w((128, MD)), w((1, MD)),
                  w((MD, MD)), w((1, MD)), w((128, MD)), w((1, MD)),
                  w((MD, 1)), w((1, 1))],
        out_specs=[col, m64, m64],
        out_shape=[
            jax.ShapeDtypeStruct((B, 1), jnp.float32),
            jax.ShapeDtypeStruct((B, MD), jnp.float32),
            jax.ShapeDtypeStruct((B, MD), jnp.float32),
        ],
    )(ts2, lus2, lud2, rs2, rd2, gs, gd, sf, df,
      Wn, bn2, Wp1, bp12, Wp2, bp22, Wl1, bl12, Wl2, bl22)


def kernel(src_ids, dst_ids, src_features, dst_features, timestamps,
           edge_features, memory, last_update, Wn, bn, We, be, Wt, bt,
           Wp1, bp1, Wp2, bp2, Wl1, bl1, Wl2, bl2):
    sid = src_ids.astype(jnp.int32)
    did = dst_ids.astype(jnp.int32)
    spair = jnp.where(sid < NPAIR, sid, sid - NPAIR).reshape(IDX_ROWS, CHUNK)
    dpair = jnp.where(did < NPAIR, did, did - NPAIR).reshape(IDX_ROWS, CHUNK)
    sidx = sid.reshape(IDX_ROWS, CHUNK)
    didx = did.reshape(IDX_ROWS, CHUNK)
    memL = _repack(memory)
    gs, gd, lus, lud = _sc_gather(memL, last_update, spair, dpair, sidx, didx)
    ts2 = timestamps.reshape(B, 1)
    rs2 = (sid >= NPAIR).astype(jnp.int32).reshape(B, 1)
    rd2 = (did >= NPAIR).astype(jnp.int32).reshape(B, 1)
    link, semb, demb = _tc_dense(
        ts2, lus.reshape(B, 1), lud.reshape(B, 1), rs2, rd2, gs, gd,
        src_features, dst_features,
        Wn, bn.reshape(1, MD), Wp1, bp1.reshape(1, MD), Wp2, bp2.reshape(1, MD),
        Wl1, bl1.reshape(1, MD), Wl2, bl2.reshape(1, 1))
    return link, semb, demb
